# baseline (speedup 1.0000x reference)
import functools

import jax
import jax.numpy as jnp
from jax import lax
from jax.experimental import pallas as pl
from jax.experimental.pallas import tpu as pltpu
from jax.experimental.pallas import tpu_sc as plsc

B = 4096
D = 64
SUB = 8
NUM_CORES = 2
NUM_SUBCORES = 16
LANES = 16
NW = NUM_CORES * NUM_SUBCORES
BPW = B // NW
GROUPS = BPW // LANES
UNROLL = 4

_mesh = plsc.VectorSubcoreMesh(core_axis_name="c", subcore_axis_name="s")


@functools.partial(
    pl.kernel,
    out_type=jax.ShapeDtypeStruct((B,), jnp.float32),
    mesh=_mesh,
    compiler_params=pltpu.CompilerParams(needs_layout_passes=False),
    scratch_types=[
        pltpu.VMEM((BPW,), jnp.int32),
        pltpu.VMEM((BPW,), jnp.int32),
        pltpu.VMEM((BPW,), jnp.int32),
        pltpu.VMEM((BPW, D), jnp.float32),
        pltpu.VMEM((BPW, D), jnp.float32),
        pltpu.VMEM((BPW, D), jnp.float32),
        pltpu.VMEM((BPW,), jnp.float32),
        pltpu.SemaphoreType.DMA,
        pltpu.SemaphoreType.DMA,
        pltpu.SemaphoreType.DMA,
        pltpu.SemaphoreType.DMA,
    ],
)
def _l1_score_kernel(heads, rels, tails, etab, rtab, out,
                     hraw, rraw, traw,
                     hrows, rrows, trows, sums,
                     sem_h, sem_r, sem_t, sem_i):
    wid = lax.axis_index("s") * NUM_CORES + lax.axis_index("c")
    base = wid * BPW
    bsl = pl.ds(base, BPW)

    c1 = pltpu.async_copy(heads.at[bsl], hraw, sem_i)
    c2 = pltpu.async_copy(rels.at[bsl], rraw, sem_i)
    c3 = pltpu.async_copy(tails.at[bsl], traw, sem_i)
    c1.wait()
    c2.wait()
    c3.wait()

    for g in range(GROUPS):
        sl = pl.ds(g * LANES, LANES)
        hv = hraw[sl]
        rv = rraw[sl]
        tv = traw[sl]
        for j in range(LANES):
            i = g * LANES + j
            pltpu.async_copy(
                etab.at[lax.shift_right_logical(hv[j], 3),
                        lax.bitwise_and(hv[j], 7)],
                hrows.at[i], sem_h)
            pltpu.async_copy(rtab.at[rv[j]], rrows.at[i], sem_r)
            pltpu.async_copy(
                etab.at[lax.shift_right_logical(tv[j], 3),
                        lax.bitwise_and(tv[j], 7)],
                trows.at[i], sem_t)

    dummy = etab.at[0, 0]
    rdummy = rtab.at[0]

    def drain(i, _):
        pltpu.make_async_copy(dummy, hrows.at[i], sem_h).wait()
        pltpu.make_async_copy(rdummy, rrows.at[i], sem_r).wait()
        pltpu.make_async_copy(dummy, trows.at[i], sem_t).wait()
        return 0

    lax.fori_loop(0, BPW, drain, 0)

    lanes = lax.iota(jnp.int32, LANES)
    for g in range(GROUPS):
        sl = pl.ds(g * LANES, LANES)
        rows = lanes + g * LANES

        def body(kk, acc):
            d0 = kk * UNROLL
            for u in range(UNROLL):
                col = jnp.full((LANES,), d0 + u, dtype=jnp.int32)
                h = plsc.load_gather(hrows, [rows, col])
                r = plsc.load_gather(rrows, [rows, col])
                t = plsc.load_gather(trows, [rows, col])
                acc = acc + jnp.abs(h + r - t)
            return acc

        acc = lax.fori_loop(0, D // UNROLL, body,
                            jnp.zeros((LANES,), jnp.float32))
        sums[sl] = acc

    pltpu.sync_copy(sums, out.at[pl.ds(base, BPW)])


def kernel(sample, entity_embedding, relation_embedding, gamma):
    heads = sample[:, 0]
    rels = sample[:, 1]
    tails = sample[:, 2]
    etab3 = entity_embedding.reshape(-1, SUB, D)
    sums = _l1_score_kernel(heads, rels, tails, etab3, relation_embedding)
    return (gamma - sums)[:, None]
